# baseline (speedup 1.0000x reference)
import functools

import jax
import jax.numpy as jnp
from jax import lax
from jax.experimental import pallas as pl
from jax.experimental.pallas import tpu as pltpu
from jax.experimental.pallas import tpu_sc as plsc

KCODES = 8192
DIM = 64
NTOK = 8192
TM = 512
KB = 2048
NT = NTOK // TM
NKB = KCODES // KB


def _argmin_body(z_ref, z2_ref, cb_ref, c2_ref, idx_ref, commit_ref,
                 rmin_ref, ridx_ref, rdist_ref):
    t = pl.program_id(0)
    k = pl.program_id(1)
    z = z_ref[...]
    cb = cb_ref[...]
    mm = lax.dot_general(z, cb, (((1,), (1,)), ((), ())),
                         preferred_element_type=jnp.float32)
    dist = z2_ref[...] - 2.0 * mm + c2_ref[...]
    bmin = jnp.min(dist, axis=1, keepdims=True)
    iota = lax.broadcasted_iota(jnp.int32, (TM, KB), 1)
    cand = jnp.where(dist == bmin, iota, KB)
    bidx = jnp.min(cand, axis=1, keepdims=True) + k * KB
    bmin_b = bmin.astype(jnp.bfloat16).astype(jnp.float32)

    @pl.when(k == 0)
    def _():
        rmin_ref[...] = bmin_b
        ridx_ref[...] = bidx
        rdist_ref[...] = bmin

    @pl.when(k > 0)
    def _():
        win = bmin < rmin_ref[...]
        ridx_ref[...] = jnp.where(win, bidx, ridx_ref[...])
        rmin_ref[...] = jnp.where(win, bmin_b, rmin_ref[...])
        rdist_ref[...] = jnp.where(win, bmin, rdist_ref[...])

    @pl.when(k == NKB - 1)
    def _():
        idx_ref[...] = ridx_ref[...]
        s = jnp.sum(rdist_ref[...], keepdims=True).reshape(1, 1)

        @pl.when(t == 0)
        def _():
            commit_ref[...] = s

        @pl.when(t > 0)
        def _():
            commit_ref[...] = commit_ref[...] + s


@jax.jit
def _dist_argmin(zb, z2, cbb, c2):
    return pl.pallas_call(
        _argmin_body,
        grid=(NT, NKB),
        in_specs=[
            pl.BlockSpec((TM, DIM), lambda t, k: (t, 0)),
            pl.BlockSpec((TM, 1), lambda t, k: (t, 0)),
            pl.BlockSpec((KB, DIM), lambda t, k: (k, 0)),
            pl.BlockSpec((1, KB), lambda t, k: (0, k)),
        ],
        out_specs=[
            pl.BlockSpec((TM, 1), lambda t, k: (t, 0)),
            pl.BlockSpec((1, 1), lambda t, k: (0, 0)),
        ],
        out_shape=[
            jax.ShapeDtypeStruct((NTOK, 1), jnp.int32),
            jax.ShapeDtypeStruct((1, 1), jnp.float32),
        ],
        scratch_shapes=[
            pltpu.VMEM((TM, 1), jnp.float32),
            pltpu.VMEM((TM, 1), jnp.int32),
            pltpu.VMEM((TM, 1), jnp.float32),
        ],
        compiler_params=pltpu.CompilerParams(
            dimension_semantics=("arbitrary", "arbitrary")),
    )(zb, z2, cbb, c2)


_NW = 32
_BPW = NTOK // _NW
_CH = 128
_NCH = _BPW // _CH
_GW = 128


@jax.jit
def _sc_gather(codebook_pad, idx2d):
    mesh = plsc.VectorSubcoreMesh(core_axis_name="c", subcore_axis_name="s")

    @functools.partial(
        pl.kernel, mesh=mesh,
        out_type=jax.ShapeDtypeStruct((NTOK, _GW), jnp.float32),
        scratch_types=[
            pltpu.VMEM((_NCH, _CH), jnp.int32),
            pltpu.VMEM((_BPW, _GW), jnp.float32),
            pltpu.SemaphoreType.DMA,
        ])
    def body(cb_hbm, idx_hbm, out_hbm, idx_v, rows_v, sem):
        c = lax.axis_index("c")
        s = lax.axis_index("s")
        wid = s * 2 + c
        pltpu.sync_copy(idx_hbm.at[pl.ds(wid * _NCH, _NCH)], idx_v)
        cps = [pltpu.async_copy(cb_hbm.at[idx_v.at[j]],
                                rows_v.at[pl.ds(j * _CH, _CH)], sem)
               for j in range(_NCH)]
        for cp in cps:
            cp.wait()
        pltpu.sync_copy(rows_v, out_hbm.at[pl.ds(wid * _BPW, _BPW)])

    return body(codebook_pad, idx2d)


def kernel(x, codebook):
    z = x.reshape(-1, DIM)
    z2 = jnp.sum(z * z, axis=1, keepdims=True)
    c2 = jnp.sum(codebook * codebook, axis=1)[None, :]
    zb = z.astype(jnp.bfloat16)
    cbb = codebook.astype(jnp.bfloat16)
    idx_col, commit_sum = _dist_argmin(zb, z2, cbb, c2)
    idx = idx_col[:, 0]
    codebook_pad = jnp.pad(codebook, ((0, 0), (0, _GW - DIM)))
    z_q = _sc_gather(codebook_pad, idx.reshape(_NW * _NCH, _CH))[:, :DIM]
    qe = z + lax.stop_gradient(z_q - z)
    commit_loss = commit_sum[0, 0] / jnp.float32(NTOK * DIM)
    qe = qe.reshape(x.shape)
    indices = idx.reshape(x.shape[:-1])
    return qe, commit_loss, indices


# speedup vs baseline: 1.0694x; 1.0694x over previous
import functools

import jax
import jax.numpy as jnp
from jax import lax
from jax.experimental import pallas as pl
from jax.experimental.pallas import tpu as pltpu
from jax.experimental.pallas import tpu_sc as plsc

KCODES = 8192
DIM = 64
NTOK = 8192
TM = 1024
KB = 2048
NT = NTOK // TM
NKB = KCODES // KB


def _argmin_body(z_ref, z2_ref, cb_ref, c2_ref, idx_ref, commit_ref,
                 rmin_ref, ridx_ref, rdist_ref):
    t = pl.program_id(0)
    k = pl.program_id(1)
    z = z_ref[...]
    cb = cb_ref[...]
    mm = lax.dot_general(z, cb, (((1,), (1,)), ((), ())),
                         preferred_element_type=jnp.float32)
    dist = z2_ref[...] - 2.0 * mm + c2_ref[...]
    bmin = jnp.min(dist, axis=1, keepdims=True)
    iota = lax.broadcasted_iota(jnp.int32, (TM, KB), 1)
    cand = jnp.where(dist == bmin, iota, KB)
    bidx = jnp.min(cand, axis=1, keepdims=True) + k * KB
    bmin_b = bmin.astype(jnp.bfloat16).astype(jnp.float32)

    @pl.when(k == 0)
    def _():
        rmin_ref[...] = bmin_b
        ridx_ref[...] = bidx
        rdist_ref[...] = bmin

    @pl.when(k > 0)
    def _():
        win = bmin < rmin_ref[...]
        ridx_ref[...] = jnp.where(win, bidx, ridx_ref[...])
        rmin_ref[...] = jnp.where(win, bmin_b, rmin_ref[...])
        rdist_ref[...] = jnp.where(win, bmin, rdist_ref[...])

    @pl.when(k == NKB - 1)
    def _():
        idx_ref[...] = ridx_ref[...]
        s = jnp.sum(rdist_ref[...], keepdims=True).reshape(1, 1)

        @pl.when(t == 0)
        def _():
            commit_ref[...] = s

        @pl.when(t > 0)
        def _():
            commit_ref[...] = commit_ref[...] + s


@jax.jit
def _dist_argmin(zb, z2, cbb, c2):
    return pl.pallas_call(
        _argmin_body,
        grid=(NT, NKB),
        in_specs=[
            pl.BlockSpec((TM, DIM), lambda t, k: (t, 0)),
            pl.BlockSpec((TM, 1), lambda t, k: (t, 0)),
            pl.BlockSpec((KB, DIM), lambda t, k: (k, 0)),
            pl.BlockSpec((1, KB), lambda t, k: (0, k)),
        ],
        out_specs=[
            pl.BlockSpec((TM, 1), lambda t, k: (t, 0)),
            pl.BlockSpec((1, 1), lambda t, k: (0, 0)),
        ],
        out_shape=[
            jax.ShapeDtypeStruct((NTOK, 1), jnp.int32),
            jax.ShapeDtypeStruct((1, 1), jnp.float32),
        ],
        scratch_shapes=[
            pltpu.VMEM((TM, 1), jnp.float32),
            pltpu.VMEM((TM, 1), jnp.int32),
            pltpu.VMEM((TM, 1), jnp.float32),
        ],
        compiler_params=pltpu.CompilerParams(
            dimension_semantics=("arbitrary", "arbitrary")),
    )(zb, z2, cbb, c2)


_NW = 32
_BPW = NTOK // _NW
_CH = 128
_NCH = _BPW // _CH
_GW = 128


@jax.jit
def _sc_gather(codebook_pad, idx2d):
    mesh = plsc.VectorSubcoreMesh(core_axis_name="c", subcore_axis_name="s")

    @functools.partial(
        pl.kernel, mesh=mesh,
        out_type=jax.ShapeDtypeStruct((NTOK, _GW), jnp.float32),
        scratch_types=[
            pltpu.VMEM((_NCH, _CH), jnp.int32),
            pltpu.VMEM((_BPW, _GW), jnp.float32),
            pltpu.SemaphoreType.DMA,
        ])
    def body(cb_hbm, idx_hbm, out_hbm, idx_v, rows_v, sem):
        c = lax.axis_index("c")
        s = lax.axis_index("s")
        wid = s * 2 + c
        pltpu.sync_copy(idx_hbm.at[pl.ds(wid * _NCH, _NCH)], idx_v)
        cps = [pltpu.async_copy(cb_hbm.at[idx_v.at[j]],
                                rows_v.at[pl.ds(j * _CH, _CH)], sem)
               for j in range(_NCH)]
        for cp in cps:
            cp.wait()
        pltpu.sync_copy(rows_v, out_hbm.at[pl.ds(wid * _BPW, _BPW)])

    return body(codebook_pad, idx2d)


def kernel(x, codebook):
    z = x.reshape(-1, DIM)
    z2 = jnp.sum(z * z, axis=1, keepdims=True)
    c2 = jnp.sum(codebook * codebook, axis=1)[None, :]
    zb = z.astype(jnp.bfloat16)
    cbb = codebook.astype(jnp.bfloat16)
    idx_col, commit_sum = _dist_argmin(zb, z2, cbb, c2)
    idx = idx_col[:, 0]
    codebook_pad = jnp.pad(codebook, ((0, 0), (0, _GW - DIM)))
    z_q = _sc_gather(codebook_pad, idx.reshape(_NW * _NCH, _CH))[:, :DIM]
    qe = z + lax.stop_gradient(z_q - z)
    commit_loss = commit_sum[0, 0] / jnp.float32(NTOK * DIM)
    qe = qe.reshape(x.shape)
    indices = idx.reshape(x.shape[:-1])
    return qe, commit_loss, indices
